# baseline (speedup 1.0000x reference)
import functools

import jax
import jax.numpy as jnp
from jax import lax
from jax.experimental import pallas as pl
from jax.experimental.pallas import tpu as pltpu
from jax.experimental.pallas import tpu_sc as plsc

D = 64
B = 16384
NC, NS, L = 2, 16, 16
NW = NC * NS
BPW = B // NW
CHUNK = 128
NCH = BPW // CHUNK

_mesh = plsc.VectorSubcoreMesh(core_axis_name="c", subcore_axis_name="s")


@functools.partial(
    pl.kernel,
    out_type=jax.ShapeDtypeStruct((NW, L), jnp.float32),
    mesh=_mesh,
    compiler_params=pltpu.CompilerParams(use_tc_tiling_on_sc=False),
    scratch_types=[
        pltpu.VMEM((NCH, CHUNK), jnp.int32),
        pltpu.VMEM((BPW, D), jnp.float32),
        pltpu.VMEM((BPW, D), jnp.float32),
        pltpu.VMEM((L,), jnp.float32),
        pltpu.SemaphoreType.DMA,
        pltpu.SemaphoreType.DMA,
    ],
)
def _sc_partials(feat_hbm, lab_hbm, cent_hbm, out_hbm,
                 idx_v, feat_v, rows_v, acc_v, gsem, fsem):
    wid = lax.axis_index("s") * NC + lax.axis_index("c")
    base = wid * BPW
    pltpu.sync_copy(lab_hbm.at[wid], idx_v)
    fcp = pltpu.async_copy(feat_hbm.at[pl.ds(base, BPW)], feat_v, fsem)
    gcps = [
        pltpu.async_copy(cent_hbm.at[idx_v.at[j]],
                         rows_v.at[pl.ds(j * CHUNK, CHUNK)], gsem)
        for j in range(NCH)
    ]
    fcp.wait()
    for g in gcps:
        g.wait()

    def body(r, accs):
        out = []
        for c in range(D // L):
            f = feat_v[r, pl.ds(c * L, L)]
            g = rows_v[r, pl.ds(c * L, L)]
            dlt = f - g
            out.append(accs[c] + dlt * dlt)
        return tuple(out)

    zero = jnp.zeros((L,), jnp.float32)
    accs = lax.fori_loop(0, BPW, body, (zero,) * (D // L))
    acc_v[...] = (accs[0] + accs[1]) + (accs[2] + accs[3])
    pltpu.sync_copy(acc_v, out_hbm.at[wid])


def _tc_mean_body(p_ref, o_ref):
    o_ref[0, 0] = jnp.sum(p_ref[...]) * (1.0 / B)


_tc_mean = pl.pallas_call(
    _tc_mean_body,
    out_shape=jax.ShapeDtypeStruct((1, 1), jnp.float32),
    out_specs=pl.BlockSpec(memory_space=pltpu.SMEM),
)


def kernel(features, labels, centers):
    lab = labels.astype(jnp.int32).reshape(NW, NCH, CHUNK)
    partials = _sc_partials(features, lab, centers)
    return _tc_mean(partials)[0, 0]
